# Initial kernel scaffold; baseline (speedup 1.0000x reference)
import jax
import jax.numpy as jnp
from jax.experimental import pallas as pl


def kernel(e1__w1, e1__b1, e1__g1, e1__bt1, e1__w2, e1__b2, e1__g2, e1__bt2, e2__w1, e2__b1, e2__g1, e2__bt1, e2__w2, e2__b2, e2__g2, e2__bt2, e3__w1, e3__b1, e3__g1, e3__bt1, e3__w2, e3__b2, e3__g2, e3__bt2, e4__w1, e4__b1, e4__g1, e4__bt1, e4__w2, e4__b2, e4__g2, e4__bt2, b__w1, b__b1, b__g1, b__bt1, b__w2, b__b2, b__g2, b__bt2, d1__w1, d1__b1, d1__g1, d1__bt1, d1__w2, d1__b2, d1__g2, d1__bt2, d1__up_w, d1__up_b, d2__w1, d2__b1, d2__g1, d2__bt1, d2__w2, d2__b2, d2__g2, d2__bt2, d2__up_w, d2__up_b, d3__w1, d3__b1, d3__g1, d3__bt1, d3__w2, d3__b2, d3__g2, d3__bt2, d3__up_w, d3__up_b, d4__w1, d4__b1, d4__g1, d4__bt1, d4__w2, d4__b2, d4__g2, d4__bt2, d4__up_w, d4__up_b, cls_w, cls_b, x):
    raise NotImplementedError("write your pallas kernel here")


# speedup vs baseline: 2.4614x; 2.4614x over previous
import functools

import jax
import jax.numpy as jnp
from jax.experimental import pallas as pl
from jax.experimental.pallas import tpu as pltpu

_LANE = 128
_BN_EPS = 1e-5
_VMEM_LIMIT = 56 * 1024 * 1024


def _rup(x, m):
    return (x + m - 1) // m * m


def _cparams(dim_sem):
    return pltpu.CompilerParams(dimension_semantics=dim_sem,
                                vmem_limit_bytes=_VMEM_LIMIT)


def _dconv_body(*refs, nsrc, th, w, h, bands_per_img):
    src_refs = refs[:nsrc]
    w_ref = refs[nsrc]
    y_ref = refs[nsrc + 1]
    s_ref = refs[nsrc + 2]

    i = pl.program_id(0)
    n = i // bands_per_img
    hb = i % bands_per_img
    r0 = n * (h + 2) + hb * th

    pieces = []
    for kh in range(3):
        rows = [r[pl.ds(r0 + kh, th), :, :] for r in src_refs]
        for kw in range(3):
            for rs in rows:
                pieces.append(rs[:, kw:kw + w, :])
    patch = jnp.concatenate(pieces, axis=-1)
    patch = patch.reshape(th * w, patch.shape[-1])

    y = jnp.dot(patch, w_ref[...], preferred_element_type=jnp.float32)
    y_ref[...] = y

    s = jnp.sum(y, axis=0, keepdims=True)
    ss = jnp.sum(y * y, axis=0, keepdims=True)
    ridx = jax.lax.broadcasted_iota(jnp.int32, s_ref.shape, 0)
    s_ref[...] = jnp.where(ridx == 0, jnp.broadcast_to(s, s_ref.shape),
                           jnp.where(ridx == 1,
                                     jnp.broadcast_to(ss, s_ref.shape), 0.0))


def _direct_conv_stats(xs, w_hwio):
    N, H, W, _ = xs[0].shape
    Cin = sum(t.shape[-1] for t in xs)
    Cout = w_hwio.shape[-1]
    Cp = _rup(max(Cout, _LANE), _LANE)
    M = N * H * W

    th = min(H, max(1, 512 // W))
    while H % th:
        th -= 1
    bands_per_img = H // th
    nb = N * bands_per_img

    xp = [jnp.pad(t.astype(jnp.bfloat16),
                  ((0, 0), (1, 1), (1, 1), (0, 0)))
          .reshape(N * (H + 2), W + 2, t.shape[-1]) for t in xs]
    wmat = w_hwio.reshape(9 * Cin, Cout)
    if Cp != Cout:
        wmat = jnp.pad(wmat, ((0, 0), (0, Cp - Cout)))
    wmat = wmat.astype(jnp.bfloat16)

    body = functools.partial(_dconv_body, nsrc=len(xs), th=th, w=W, h=H,
                             bands_per_img=bands_per_img)

    in_specs = [pl.BlockSpec(t.shape, lambda i: (0, 0, 0)) for t in xp]
    in_specs.append(pl.BlockSpec(wmat.shape, lambda i: (0, 0)))

    y, stats = pl.pallas_call(
        body,
        out_shape=(jax.ShapeDtypeStruct((M, Cp), jnp.float32),
                   jax.ShapeDtypeStruct((nb * 8, Cp), jnp.float32)),
        grid=(nb,),
        in_specs=in_specs,
        out_specs=(pl.BlockSpec((th * W, Cp), lambda i: (i, 0)),
                   pl.BlockSpec((8, Cp), lambda i: (i, 0))),
        compiler_params=_cparams(("parallel",)),
    )(*xp, wmat)
    return y, stats


def _mm_body(*refs, with_bias, with_stats, out_dtype):
    x_ref, w_ref = refs[0], refs[1]
    idx = 2 + (1 if with_bias else 0)
    y_ref = refs[idx]
    y = jnp.dot(x_ref[...], w_ref[...], preferred_element_type=jnp.float32)
    if with_bias:
        y = y + refs[2][...]
    y_ref[...] = y.astype(out_dtype)
    if with_stats:
        s_ref = refs[idx + 1]
        s = jnp.sum(y, axis=0, keepdims=True)
        ss = jnp.sum(y * y, axis=0, keepdims=True)
        ridx = jax.lax.broadcasted_iota(jnp.int32, s_ref.shape, 0)
        s_ref[...] = jnp.where(ridx == 0, jnp.broadcast_to(s, s_ref.shape),
                               jnp.where(ridx == 1,
                                         jnp.broadcast_to(ss, s_ref.shape),
                                         0.0))


def _matmul(x, wmat, bias=None, with_stats=False, out_dtype=jnp.float32):
    M, K = x.shape
    C = wmat.shape[1]
    Cp = _rup(max(C, _LANE), _LANE)
    if Cp != C:
        wmat = jnp.pad(wmat, ((0, 0), (0, Cp - C)))
        if bias is not None:
            bias = jnp.pad(bias, ((0, Cp - C),))
    wmat = wmat.astype(jnp.bfloat16)

    tm = min(M, 512)
    while M % tm:
        tm -= 8
    mt = M // tm

    ops = [x.astype(jnp.bfloat16), wmat]
    in_specs = [pl.BlockSpec((tm, K), lambda i: (i, 0)),
                pl.BlockSpec((K, Cp), lambda i: (0, 0))]
    if bias is not None:
        ops.append(bias.reshape(1, Cp).astype(jnp.float32))
        in_specs.append(pl.BlockSpec((1, Cp), lambda i: (0, 0)))

    out_shape = [jax.ShapeDtypeStruct((M, Cp), out_dtype)]
    out_specs = [pl.BlockSpec((tm, Cp), lambda i: (i, 0))]
    if with_stats:
        out_shape.append(jax.ShapeDtypeStruct((mt * 8, Cp), jnp.float32))
        out_specs.append(pl.BlockSpec((8, Cp), lambda i: (i, 0)))

    body = functools.partial(_mm_body, with_bias=bias is not None,
                             with_stats=with_stats, out_dtype=out_dtype)
    res = pl.pallas_call(
        body,
        out_shape=tuple(out_shape),
        grid=(mt,),
        in_specs=in_specs,
        out_specs=tuple(out_specs),
        compiler_params=_cparams(("parallel",)),
    )(*ops)
    return res if with_stats else res[0]


def _bn_body(y_ref, sc_ref, sh_ref, o_ref):
    o_ref[...] = jnp.maximum(
        y_ref[...] * sc_ref[...] + sh_ref[...], 0.0).astype(o_ref.dtype)


def _bn_relu(y, scale, shift):
    M, Cp = y.shape
    tm = min(M, 512)
    while M % tm:
        tm -= 8
    return pl.pallas_call(
        _bn_body,
        out_shape=jax.ShapeDtypeStruct((M, Cp), jnp.bfloat16),
        grid=(M // tm,),
        in_specs=[pl.BlockSpec((tm, Cp), lambda i: (i, 0)),
                  pl.BlockSpec((1, Cp), lambda i: (0, 0)),
                  pl.BlockSpec((1, Cp), lambda i: (0, 0))],
        out_specs=pl.BlockSpec((tm, Cp), lambda i: (i, 0)),
        compiler_params=_cparams(("parallel",)),
    )(y, scale.reshape(1, Cp), shift.reshape(1, Cp))


def _finish_bn(y, stats, M, Cout, gamma, beta):
    Cp = y.shape[1]
    st = stats.reshape(-1, 8, Cp)
    total = jnp.sum(st[:, 0, :], axis=0)
    total_sq = jnp.sum(st[:, 1, :], axis=0)
    mean = total / M
    var = jnp.maximum(total_sq / M - mean * mean, 0.0)
    inv = jax.lax.rsqrt(var + _BN_EPS)
    gamma_p = jnp.pad(gamma.astype(jnp.float32), (0, Cp - Cout))
    beta_p = jnp.pad(beta.astype(jnp.float32), (0, Cp - Cout))
    scale = gamma_p * inv
    shift = beta_p - mean * scale
    return _bn_relu(y, scale, shift)


def _conv_bn_relu(xs, w_hwio, gamma, beta):
    if not isinstance(xs, (list, tuple)):
        xs = [xs]
    N, H, W, _ = xs[0].shape
    Cout = w_hwio.shape[-1]
    M = N * H * W
    y, stats = _direct_conv_stats(xs, w_hwio)
    out = _finish_bn(y, stats, M, Cout, gamma, beta)
    return out[:, :Cout].reshape(N, H, W, Cout)


def _first_conv_bn_relu(x, w_hwio, gamma, beta):
    N, H, W, Cin = x.shape
    Cout = w_hwio.shape[-1]
    M = N * H * W
    xp = jnp.pad(x.astype(jnp.bfloat16), ((0, 0), (1, 1), (1, 1), (0, 0)))
    cols = [xp[:, kh:kh + H, kw:kw + W, :]
            for kh in range(3) for kw in range(3)]
    patches = jnp.concatenate(cols, axis=-1).reshape(M, 9 * Cin)
    y, stats = _matmul(patches, w_hwio.reshape(9 * Cin, Cout),
                       with_stats=True)
    out = _finish_bn(y, stats, M, Cout, gamma, beta)
    return out[:, :Cout].reshape(N, H, W, Cout)


def _double_conv(p, xs, first=False):
    if first:
        x = _first_conv_bn_relu(xs, p["w1"], p["g1"], p["bt1"])
    else:
        x = _conv_bn_relu(xs, p["w1"], p["g1"], p["bt1"])
    return _conv_bn_relu(x, p["w2"], p["g2"], p["bt2"])


def _maxpool2(x):
    N, H, W, C = x.shape
    return x.reshape(N, H // 2, 2, W // 2, 2, C).max(axis=(2, 4))


def _up_conv(x, w, b):
    N, H, W, Cin = x.shape
    Cout = w.shape[-1]
    wmat = w.reshape(Cin, 4 * Cout)
    y = _matmul(x.reshape(N * H * W, Cin), wmat, bias=jnp.tile(b, 4),
                out_dtype=jnp.bfloat16)
    y = y[:, :4 * Cout].reshape(N, H, W, 2, 2, Cout)
    y = jnp.transpose(y, (0, 1, 3, 2, 4, 5)).reshape(N, 2 * H, 2 * W, Cout)
    return y


def kernel(e1__w1, e1__b1, e1__g1, e1__bt1, e1__w2, e1__b2, e1__g2, e1__bt2, e2__w1, e2__b1, e2__g1, e2__bt1, e2__w2, e2__b2, e2__g2, e2__bt2, e3__w1, e3__b1, e3__g1, e3__bt1, e3__w2, e3__b2, e3__g2, e3__bt2, e4__w1, e4__b1, e4__g1, e4__bt1, e4__w2, e4__b2, e4__g2, e4__bt2, b__w1, b__b1, b__g1, b__bt1, b__w2, b__b2, b__g2, b__bt2, d1__w1, d1__b1, d1__g1, d1__bt1, d1__w2, d1__b2, d1__g2, d1__bt2, d1__up_w, d1__up_b, d2__w1, d2__b1, d2__g1, d2__bt1, d2__w2, d2__b2, d2__g2, d2__bt2, d2__up_w, d2__up_b, d3__w1, d3__b1, d3__g1, d3__bt1, d3__w2, d3__b2, d3__g2, d3__bt2, d3__up_w, d3__up_b, d4__w1, d4__b1, d4__g1, d4__bt1, d4__w2, d4__b2, d4__g2, d4__bt2, d4__up_w, d4__up_b, cls_w, cls_b, x):
    p = {
        "e1": dict(w1=e1__w1, g1=e1__g1, bt1=e1__bt1,
                   w2=e1__w2, g2=e1__g2, bt2=e1__bt2),
        "e2": dict(w1=e2__w1, g1=e2__g1, bt1=e2__bt1,
                   w2=e2__w2, g2=e2__g2, bt2=e2__bt2),
        "e3": dict(w1=e3__w1, g1=e3__g1, bt1=e3__bt1,
                   w2=e3__w2, g2=e3__g2, bt2=e3__bt2),
        "e4": dict(w1=e4__w1, g1=e4__g1, bt1=e4__bt1,
                   w2=e4__w2, g2=e4__g2, bt2=e4__bt2),
        "b": dict(w1=b__w1, g1=b__g1, bt1=b__bt1,
                  w2=b__w2, g2=b__g2, bt2=b__bt2),
        "d1": dict(w1=d1__w1, g1=d1__g1, bt1=d1__bt1,
                   w2=d1__w2, g2=d1__g2, bt2=d1__bt2,
                   up_w=d1__up_w, up_b=d1__up_b),
        "d2": dict(w1=d2__w1, g1=d2__g1, bt1=d2__bt1,
                   w2=d2__w2, g2=d2__g2, bt2=d2__bt2,
                   up_w=d2__up_w, up_b=d2__up_b),
        "d3": dict(w1=d3__w1, g1=d3__g1, bt1=d3__bt1,
                   w2=d3__w2, g2=d3__g2, bt2=d3__bt2,
                   up_w=d3__up_w, up_b=d3__up_b),
        "d4": dict(w1=d4__w1, g1=d4__g1, bt1=d4__bt1,
                   w2=d4__w2, g2=d4__g2, bt2=d4__bt2,
                   up_w=d4__up_w, up_b=d4__up_b),
    }
    xin = jnp.transpose(x, (0, 2, 3, 1))

    e1 = _double_conv(p["e1"], xin, first=True)
    p1 = _maxpool2(e1)
    e2 = _double_conv(p["e2"], p1)
    p2 = _maxpool2(e2)
    e3 = _double_conv(p["e3"], p2)
    p3 = _maxpool2(e3)
    e4 = _double_conv(p["e4"], p3)
    p4 = _maxpool2(e4)
    bt = _double_conv(p["b"], p4)

    d = bt
    for name, skip in (("d1", e4), ("d2", e3), ("d3", e2), ("d4", e1)):
        u = _up_conv(d, p[name]["up_w"], p[name]["up_b"])
        d = _double_conv(p[name], [u, skip])

    N, H, W, C = d.shape
    logits = _matmul(d.reshape(N * H * W, C), cls_w, bias=cls_b)
    out = logits[:, :1].reshape(N, H, W, 1).astype(jnp.float32)
    return jnp.transpose(out, (0, 3, 1, 2))


# speedup vs baseline: 2.4964x; 1.0142x over previous
import functools

import jax
import jax.numpy as jnp
from jax.experimental import pallas as pl
from jax.experimental.pallas import tpu as pltpu

_LANE = 128
_BN_EPS = 1e-5
_VMEM_LIMIT = 56 * 1024 * 1024


def _rup(x, m):
    return (x + m - 1) // m * m


def _cparams(dim_sem):
    return pltpu.CompilerParams(dimension_semantics=dim_sem,
                                vmem_limit_bytes=_VMEM_LIMIT)


def _dconv_body(*refs, nsrc, th, w, h, bands_per_img):
    src_refs = refs[:nsrc]
    w_ref = refs[nsrc]
    y_ref = refs[nsrc + 1]
    s_ref = refs[nsrc + 2]

    i = pl.program_id(0)
    n = i // bands_per_img
    hb = i % bands_per_img
    r0 = n * (h + 2) + hb * th

    pieces = []
    for kh in range(3):
        rows = [r[pl.ds(r0 + kh, th), :, :] for r in src_refs]
        for kw in range(3):
            for rs in rows:
                pieces.append(rs[:, kw:kw + w, :])
    patch = jnp.concatenate(pieces, axis=-1)
    patch = patch.reshape(th * w, patch.shape[-1])

    y = jnp.dot(patch, w_ref[...], preferred_element_type=jnp.float32)
    y_ref[...] = y.astype(y_ref.dtype)

    s = jnp.sum(y, axis=0, keepdims=True)
    ss = jnp.sum(y * y, axis=0, keepdims=True)
    ridx = jax.lax.broadcasted_iota(jnp.int32, s_ref.shape, 0)
    s_ref[...] = jnp.where(ridx == 0, jnp.broadcast_to(s, s_ref.shape),
                           jnp.where(ridx == 1,
                                     jnp.broadcast_to(ss, s_ref.shape), 0.0))


def _direct_conv_stats(xs, w_hwio):
    N, H, W, _ = xs[0].shape
    Cin = sum(t.shape[-1] for t in xs)
    Cout = w_hwio.shape[-1]
    M = N * H * W

    th = min(H, max(1, 512 // W))
    while H % th:
        th -= 1
    bands_per_img = H // th
    nb = N * bands_per_img

    xp = [jnp.pad(t.astype(jnp.bfloat16),
                  ((0, 0), (1, 1), (1, 1), (0, 0)))
          .reshape(N * (H + 2), W + 2, t.shape[-1]) for t in xs]
    wmat = w_hwio.reshape(9 * Cin, Cout).astype(jnp.bfloat16)

    body = functools.partial(_dconv_body, nsrc=len(xs), th=th, w=W, h=H,
                             bands_per_img=bands_per_img)

    in_specs = [pl.BlockSpec(t.shape, lambda i: (0, 0, 0)) for t in xp]
    in_specs.append(pl.BlockSpec(wmat.shape, lambda i: (0, 0)))

    y, stats = pl.pallas_call(
        body,
        out_shape=(jax.ShapeDtypeStruct((M, Cout), jnp.float32),
                   jax.ShapeDtypeStruct((nb * 8, Cout), jnp.float32)),
        grid=(nb,),
        in_specs=in_specs,
        out_specs=(pl.BlockSpec((th * W, Cout), lambda i: (i, 0)),
                   pl.BlockSpec((8, Cout), lambda i: (i, 0))),
        compiler_params=_cparams(("parallel",)),
    )(*xp, wmat)
    return y, stats


def _mm_body(*refs, with_bias, with_stats, out_dtype):
    x_ref, w_ref = refs[0], refs[1]
    idx = 2 + (1 if with_bias else 0)
    y_ref = refs[idx]
    y = jnp.dot(x_ref[...], w_ref[...], preferred_element_type=jnp.float32)
    if with_bias:
        y = y + refs[2][...]
    y_ref[...] = y.astype(out_dtype)
    if with_stats:
        s_ref = refs[idx + 1]
        s = jnp.sum(y, axis=0, keepdims=True)
        ss = jnp.sum(y * y, axis=0, keepdims=True)
        ridx = jax.lax.broadcasted_iota(jnp.int32, s_ref.shape, 0)
        s_ref[...] = jnp.where(ridx == 0, jnp.broadcast_to(s, s_ref.shape),
                               jnp.where(ridx == 1,
                                         jnp.broadcast_to(ss, s_ref.shape),
                                         0.0))


def _matmul(x, wmat, bias=None, with_stats=False, out_dtype=jnp.float32):
    M, K = x.shape
    C = wmat.shape[1]
    Cp = C if C >= 64 else _rup(max(C, _LANE), _LANE)
    if Cp != C:
        wmat = jnp.pad(wmat, ((0, 0), (0, Cp - C)))
        if bias is not None:
            bias = jnp.pad(bias, ((0, Cp - C),))
    wmat = wmat.astype(jnp.bfloat16)

    tm = min(M, 512)
    while M % tm:
        tm -= 8
    mt = M // tm

    ops = [x.astype(jnp.bfloat16), wmat]
    in_specs = [pl.BlockSpec((tm, K), lambda i: (i, 0)),
                pl.BlockSpec((K, Cp), lambda i: (0, 0))]
    if bias is not None:
        ops.append(bias.reshape(1, Cp).astype(jnp.float32))
        in_specs.append(pl.BlockSpec((1, Cp), lambda i: (0, 0)))

    out_shape = [jax.ShapeDtypeStruct((M, Cp), out_dtype)]
    out_specs = [pl.BlockSpec((tm, Cp), lambda i: (i, 0))]
    if with_stats:
        out_shape.append(jax.ShapeDtypeStruct((mt * 8, Cp), jnp.float32))
        out_specs.append(pl.BlockSpec((8, Cp), lambda i: (i, 0)))

    body = functools.partial(_mm_body, with_bias=bias is not None,
                             with_stats=with_stats, out_dtype=out_dtype)
    res = pl.pallas_call(
        body,
        out_shape=tuple(out_shape),
        grid=(mt,),
        in_specs=in_specs,
        out_specs=tuple(out_specs),
        compiler_params=_cparams(("parallel",)),
    )(*ops)
    return res if with_stats else res[0]


def _bn_body(y_ref, sc_ref, sh_ref, o_ref, *pool_ref, wdim):
    o = jnp.maximum(
        y_ref[...].astype(jnp.float32) * sc_ref[...] + sh_ref[...],
        0.0).astype(o_ref.dtype)
    o_ref[...] = o
    if pool_ref:
        tm, c = o.shape
        o4 = o.reshape(tm // (2 * wdim), 2, wdim // 2, 2, c)
        pool_ref[0][...] = jnp.max(o4, axis=(1, 3)).reshape(tm // 4, c)


def _bn_relu(y, scale, shift, pool_w=None):
    M, C = y.shape
    tm = min(M, 512)
    while M % tm:
        tm -= 8
    outs = [jax.ShapeDtypeStruct((M, C), jnp.bfloat16)]
    out_specs = [pl.BlockSpec((tm, C), lambda i: (i, 0))]
    if pool_w is not None:
        outs.append(jax.ShapeDtypeStruct((M // 4, C), jnp.bfloat16))
        out_specs.append(pl.BlockSpec((tm // 4, C), lambda i: (i, 0)))
    res = pl.pallas_call(
        functools.partial(_bn_body, wdim=pool_w or 0),
        out_shape=tuple(outs),
        grid=(M // tm,),
        in_specs=[pl.BlockSpec((tm, C), lambda i: (i, 0)),
                  pl.BlockSpec((1, C), lambda i: (0, 0)),
                  pl.BlockSpec((1, C), lambda i: (0, 0))],
        out_specs=tuple(out_specs),
        compiler_params=_cparams(("parallel",)),
    )(y, scale.reshape(1, C), shift.reshape(1, C))
    return res


def _finish_bn(y, stats, M, gamma, beta, pool_w=None):
    C = y.shape[1]
    st = stats.reshape(-1, 8, C)
    total = jnp.sum(st[:, 0, :], axis=0)
    total_sq = jnp.sum(st[:, 1, :], axis=0)
    mean = total / M
    var = jnp.maximum(total_sq / M - mean * mean, 0.0)
    inv = jax.lax.rsqrt(var + _BN_EPS)
    scale = gamma.astype(jnp.float32) * inv
    shift = beta.astype(jnp.float32) - mean * scale
    return _bn_relu(y, scale, shift, pool_w=pool_w)


def _conv_bn_relu(xs, w_hwio, gamma, beta, pool=False):
    if not isinstance(xs, (list, tuple)):
        xs = [xs]
    N, H, W, _ = xs[0].shape
    Cout = w_hwio.shape[-1]
    M = N * H * W
    y, stats = _direct_conv_stats(xs, w_hwio)
    res = _finish_bn(y, stats, M, gamma, beta, pool_w=W if pool else None)
    if pool:
        return (res[0].reshape(N, H, W, Cout),
                res[1].reshape(N, H // 2, W // 2, Cout))
    return res[0].reshape(N, H, W, Cout)


def _first_conv_bn_relu(x, w_hwio, gamma, beta):
    N, H, W, Cin = x.shape
    Cout = w_hwio.shape[-1]
    M = N * H * W
    xp = jnp.pad(x.astype(jnp.bfloat16), ((0, 0), (1, 1), (1, 1), (0, 0)))
    cols = [xp[:, kh:kh + H, kw:kw + W, :]
            for kh in range(3) for kw in range(3)]
    patches = jnp.concatenate(cols, axis=-1).reshape(M, 9 * Cin)
    y, stats = _matmul(patches, w_hwio.reshape(9 * Cin, Cout),
                       with_stats=True)
    out = _finish_bn(y, stats, M, gamma, beta)
    return out[0].reshape(N, H, W, Cout)


def _double_conv(p, xs, first=False, pool=False):
    if first:
        x = _first_conv_bn_relu(xs, p["w1"], p["g1"], p["bt1"])
    else:
        x = _conv_bn_relu(xs, p["w1"], p["g1"], p["bt1"])
    return _conv_bn_relu(x, p["w2"], p["g2"], p["bt2"], pool=pool)


def _up_conv(x, w, b):
    N, H, W, Cin = x.shape
    Cout = w.shape[-1]
    wmat = w.reshape(Cin, 4 * Cout)
    y = _matmul(x.reshape(N * H * W, Cin), wmat, bias=jnp.tile(b, 4),
                out_dtype=jnp.bfloat16)
    y = y[:, :4 * Cout].reshape(N, H, W, 2, 2, Cout)
    y = jnp.transpose(y, (0, 1, 3, 2, 4, 5)).reshape(N, 2 * H, 2 * W, Cout)
    return y


def kernel(e1__w1, e1__b1, e1__g1, e1__bt1, e1__w2, e1__b2, e1__g2, e1__bt2, e2__w1, e2__b1, e2__g1, e2__bt1, e2__w2, e2__b2, e2__g2, e2__bt2, e3__w1, e3__b1, e3__g1, e3__bt1, e3__w2, e3__b2, e3__g2, e3__bt2, e4__w1, e4__b1, e4__g1, e4__bt1, e4__w2, e4__b2, e4__g2, e4__bt2, b__w1, b__b1, b__g1, b__bt1, b__w2, b__b2, b__g2, b__bt2, d1__w1, d1__b1, d1__g1, d1__bt1, d1__w2, d1__b2, d1__g2, d1__bt2, d1__up_w, d1__up_b, d2__w1, d2__b1, d2__g1, d2__bt1, d2__w2, d2__b2, d2__g2, d2__bt2, d2__up_w, d2__up_b, d3__w1, d3__b1, d3__g1, d3__bt1, d3__w2, d3__b2, d3__g2, d3__bt2, d3__up_w, d3__up_b, d4__w1, d4__b1, d4__g1, d4__bt1, d4__w2, d4__b2, d4__g2, d4__bt2, d4__up_w, d4__up_b, cls_w, cls_b, x):
    p = {
        "e1": dict(w1=e1__w1, g1=e1__g1, bt1=e1__bt1,
                   w2=e1__w2, g2=e1__g2, bt2=e1__bt2),
        "e2": dict(w1=e2__w1, g1=e2__g1, bt1=e2__bt1,
                   w2=e2__w2, g2=e2__g2, bt2=e2__bt2),
        "e3": dict(w1=e3__w1, g1=e3__g1, bt1=e3__bt1,
                   w2=e3__w2, g2=e3__g2, bt2=e3__bt2),
        "e4": dict(w1=e4__w1, g1=e4__g1, bt1=e4__bt1,
                   w2=e4__w2, g2=e4__g2, bt2=e4__bt2),
        "b": dict(w1=b__w1, g1=b__g1, bt1=b__bt1,
                  w2=b__w2, g2=b__g2, bt2=b__bt2),
        "d1": dict(w1=d1__w1, g1=d1__g1, bt1=d1__bt1,
                   w2=d1__w2, g2=d1__g2, bt2=d1__bt2,
                   up_w=d1__up_w, up_b=d1__up_b),
        "d2": dict(w1=d2__w1, g1=d2__g1, bt1=d2__bt1,
                   w2=d2__w2, g2=d2__g2, bt2=d2__bt2,
                   up_w=d2__up_w, up_b=d2__up_b),
        "d3": dict(w1=d3__w1, g1=d3__g1, bt1=d3__bt1,
                   w2=d3__w2, g2=d3__g2, bt2=d3__bt2,
                   up_w=d3__up_w, up_b=d3__up_b),
        "d4": dict(w1=d4__w1, g1=d4__g1, bt1=d4__bt1,
                   w2=d4__w2, g2=d4__g2, bt2=d4__bt2,
                   up_w=d4__up_w, up_b=d4__up_b),
    }
    xin = jnp.transpose(x, (0, 2, 3, 1))

    e1, p1 = _double_conv(p["e1"], xin, first=True, pool=True)
    e2, p2 = _double_conv(p["e2"], p1, pool=True)
    e3, p3 = _double_conv(p["e3"], p2, pool=True)
    e4, p4 = _double_conv(p["e4"], p3, pool=True)
    bt = _double_conv(p["b"], p4)

    d = bt
    for name, skip in (("d1", e4), ("d2", e3), ("d3", e2), ("d4", e1)):
        u = _up_conv(d, p[name]["up_w"], p[name]["up_b"])
        d = _double_conv(p[name], [u, skip])

    N, H, W, C = d.shape
    logits = _matmul(d.reshape(N * H * W, C), cls_w, bias=cls_b)
    out = logits[:, :1].reshape(N, H, W, 1).astype(jnp.float32)
    return jnp.transpose(out, (0, 3, 1, 2))


# speedup vs baseline: 3.9925x; 1.5993x over previous
import functools

import jax
import jax.numpy as jnp
from jax.experimental import pallas as pl
from jax.experimental.pallas import tpu as pltpu

_LANE = 128
_BN_EPS = 1e-5
_VMEM_LIMIT = 56 * 1024 * 1024


def _rup(x, m):
    return (x + m - 1) // m * m


def _cparams(dim_sem):
    return pltpu.CompilerParams(dimension_semantics=dim_sem,
                                vmem_limit_bytes=_VMEM_LIMIT)


def _dconv_body(*refs, nsrc, th, w, h, bands_per_img):
    src_refs = refs[:nsrc]
    w_ref = refs[nsrc]
    y_ref = refs[nsrc + 1]
    s_ref = refs[nsrc + 2]

    i = pl.program_id(0)
    n = i // bands_per_img
    hb = i % bands_per_img
    r0 = n * (h + 2) + hb * th

    pieces = []
    for kh in range(3):
        rows = [r[pl.ds(r0 + kh, th), :, :] for r in src_refs]
        for kw in range(3):
            for rs in rows:
                pieces.append(rs[:, kw:kw + w, :])
    patch = jnp.concatenate(pieces, axis=-1)
    patch = patch.reshape(th * w, patch.shape[-1])

    y = jnp.dot(patch, w_ref[...], preferred_element_type=jnp.float32)
    y_ref[...] = y.astype(y_ref.dtype)

    s = jnp.sum(y, axis=0, keepdims=True)
    ss = jnp.sum(y * y, axis=0, keepdims=True)
    ridx = jax.lax.broadcasted_iota(jnp.int32, s_ref.shape, 0)
    s_ref[...] = jnp.where(ridx == 0, jnp.broadcast_to(s, s_ref.shape),
                           jnp.where(ridx == 1,
                                     jnp.broadcast_to(ss, s_ref.shape), 0.0))


def _direct_conv_stats(xs, w_hwio):
    N, H, W, _ = xs[0].shape
    Cin = sum(t.shape[-1] for t in xs)
    Cout = w_hwio.shape[-1]
    M = N * H * W

    patch_cap = 10 * 1024 * 1024
    th = min(H, max(1, patch_cap // (W * 9 * Cin * 2)))
    while H % th:
        th -= 1
    bands_per_img = H // th
    nb = N * bands_per_img

    xp = [jnp.pad(t.astype(jnp.bfloat16),
                  ((0, 0), (1, 1), (1, 1), (0, 0)))
          .reshape(N * (H + 2), W + 2, t.shape[-1]) for t in xs]
    wmat = w_hwio.reshape(9 * Cin, Cout).astype(jnp.bfloat16)

    body = functools.partial(_dconv_body, nsrc=len(xs), th=th, w=W, h=H,
                             bands_per_img=bands_per_img)

    in_specs = [pl.BlockSpec(t.shape, lambda i: (0, 0, 0)) for t in xp]
    in_specs.append(pl.BlockSpec(wmat.shape, lambda i: (0, 0)))

    y, stats = pl.pallas_call(
        body,
        out_shape=(jax.ShapeDtypeStruct((M, Cout), jnp.float32),
                   jax.ShapeDtypeStruct((nb * 8, Cout), jnp.float32)),
        grid=(nb,),
        in_specs=in_specs,
        out_specs=(pl.BlockSpec((th * W, Cout), lambda i: (i, 0)),
                   pl.BlockSpec((8, Cout), lambda i: (i, 0))),
        compiler_params=_cparams(("parallel",)),
    )(*xp, wmat)
    return y, stats


def _mm_body(*refs, with_bias, with_stats, out_dtype):
    x_ref, w_ref = refs[0], refs[1]
    idx = 2 + (1 if with_bias else 0)
    y_ref = refs[idx]
    y = jnp.dot(x_ref[...], w_ref[...], preferred_element_type=jnp.float32)
    if with_bias:
        y = y + refs[2][...]
    y_ref[...] = y.astype(out_dtype)
    if with_stats:
        s_ref = refs[idx + 1]
        s = jnp.sum(y, axis=0, keepdims=True)
        ss = jnp.sum(y * y, axis=0, keepdims=True)
        ridx = jax.lax.broadcasted_iota(jnp.int32, s_ref.shape, 0)
        s_ref[...] = jnp.where(ridx == 0, jnp.broadcast_to(s, s_ref.shape),
                               jnp.where(ridx == 1,
                                         jnp.broadcast_to(ss, s_ref.shape),
                                         0.0))


def _matmul(x, wmat, bias=None, with_stats=False, out_dtype=jnp.float32):
    M, K = x.shape
    C = wmat.shape[1]
    Cp = C if C >= 64 else _rup(max(C, _LANE), _LANE)
    if Cp != C:
        wmat = jnp.pad(wmat, ((0, 0), (0, Cp - C)))
        if bias is not None:
            bias = jnp.pad(bias, ((0, Cp - C),))
    wmat = wmat.astype(jnp.bfloat16)

    if M >= 2048:
        tm = min(M // 4, max(512, (8 * 1024 * 1024) // (4 * Cp)))
    elif M >= 512:
        tm = M // 2
    else:
        tm = M
    while M % tm:
        tm -= 8
    mt = M // tm

    ops = [x.astype(jnp.bfloat16), wmat]
    in_specs = [pl.BlockSpec((tm, K), lambda i: (i, 0)),
                pl.BlockSpec((K, Cp), lambda i: (0, 0))]
    if bias is not None:
        ops.append(bias.reshape(1, Cp).astype(jnp.float32))
        in_specs.append(pl.BlockSpec((1, Cp), lambda i: (0, 0)))

    out_shape = [jax.ShapeDtypeStruct((M, Cp), out_dtype)]
    out_specs = [pl.BlockSpec((tm, Cp), lambda i: (i, 0))]
    if with_stats:
        out_shape.append(jax.ShapeDtypeStruct((mt * 8, Cp), jnp.float32))
        out_specs.append(pl.BlockSpec((8, Cp), lambda i: (i, 0)))

    body = functools.partial(_mm_body, with_bias=bias is not None,
                             with_stats=with_stats, out_dtype=out_dtype)
    res = pl.pallas_call(
        body,
        out_shape=tuple(out_shape),
        grid=(mt,),
        in_specs=in_specs,
        out_specs=tuple(out_specs),
        compiler_params=_cparams(("parallel",)),
    )(*ops)
    return res if with_stats else res[0]


def _bn_body(y_ref, sc_ref, sh_ref, o_ref, *pool_ref, wdim):
    o = jnp.maximum(
        y_ref[...].astype(jnp.float32) * sc_ref[...] + sh_ref[...],
        0.0).astype(o_ref.dtype)
    o_ref[...] = o
    if pool_ref:
        tm, c = o.shape
        o4 = o.reshape(tm // (2 * wdim), 2, wdim // 2, 2, c)
        pool_ref[0][...] = jnp.max(o4, axis=(1, 3)).reshape(tm // 4, c)


def _bn_relu(y, scale, shift, pool_w=None):
    M, C = y.shape
    if M >= 2048:
        tm = min(M // 4, max(512, (8 * 1024 * 1024) // (4 * C)))
    elif M >= 512:
        tm = M // 2
    else:
        tm = M
    while M % tm:
        tm -= 8
    if pool_w is not None:
        tm = max(tm, 4)
        while tm % (2 * pool_w) or M % tm:
            tm -= 8
    outs = [jax.ShapeDtypeStruct((M, C), jnp.bfloat16)]
    out_specs = [pl.BlockSpec((tm, C), lambda i: (i, 0))]
    if pool_w is not None:
        outs.append(jax.ShapeDtypeStruct((M // 4, C), jnp.bfloat16))
        out_specs.append(pl.BlockSpec((tm // 4, C), lambda i: (i, 0)))
    res = pl.pallas_call(
        functools.partial(_bn_body, wdim=pool_w or 0),
        out_shape=tuple(outs),
        grid=(M // tm,),
        in_specs=[pl.BlockSpec((tm, C), lambda i: (i, 0)),
                  pl.BlockSpec((1, C), lambda i: (0, 0)),
                  pl.BlockSpec((1, C), lambda i: (0, 0))],
        out_specs=tuple(out_specs),
        compiler_params=_cparams(("parallel",)),
    )(y, scale.reshape(1, C), shift.reshape(1, C))
    return res


def _finish_bn(y, stats, M, gamma, beta, pool_w=None):
    C = y.shape[1]
    st = stats.reshape(-1, 8, C)
    total = jnp.sum(st[:, 0, :], axis=0)
    total_sq = jnp.sum(st[:, 1, :], axis=0)
    mean = total / M
    var = jnp.maximum(total_sq / M - mean * mean, 0.0)
    inv = jax.lax.rsqrt(var + _BN_EPS)
    scale = gamma.astype(jnp.float32) * inv
    shift = beta.astype(jnp.float32) - mean * scale
    return _bn_relu(y, scale, shift, pool_w=pool_w)


def _conv_bn_relu(xs, w_hwio, gamma, beta, pool=False):
    if not isinstance(xs, (list, tuple)):
        xs = [xs]
    N, H, W, _ = xs[0].shape
    Cout = w_hwio.shape[-1]
    M = N * H * W
    y, stats = _direct_conv_stats(xs, w_hwio)
    res = _finish_bn(y, stats, M, gamma, beta, pool_w=W if pool else None)
    if pool:
        return (res[0].reshape(N, H, W, Cout),
                res[1].reshape(N, H // 2, W // 2, Cout))
    return res[0].reshape(N, H, W, Cout)


def _first_conv_bn_relu(x, w_hwio, gamma, beta):
    N, H, W, Cin = x.shape
    Cout = w_hwio.shape[-1]
    M = N * H * W
    xp = jnp.pad(x.astype(jnp.bfloat16), ((0, 0), (1, 1), (1, 1), (0, 0)))
    cols = [xp[:, kh:kh + H, kw:kw + W, :]
            for kh in range(3) for kw in range(3)]
    patches = jnp.concatenate(cols, axis=-1).reshape(M, 9 * Cin)
    y, stats = _matmul(patches, w_hwio.reshape(9 * Cin, Cout),
                       with_stats=True)
    out = _finish_bn(y, stats, M, gamma, beta)
    return out[0].reshape(N, H, W, Cout)


def _double_conv(p, xs, first=False, pool=False):
    if first:
        x = _first_conv_bn_relu(xs, p["w1"], p["g1"], p["bt1"])
    else:
        x = _conv_bn_relu(xs, p["w1"], p["g1"], p["bt1"])
    return _conv_bn_relu(x, p["w2"], p["g2"], p["bt2"], pool=pool)


def _up_conv(x, w, b):
    N, H, W, Cin = x.shape
    Cout = w.shape[-1]
    wmat = w.reshape(Cin, 4 * Cout)
    y = _matmul(x.reshape(N * H * W, Cin), wmat, bias=jnp.tile(b, 4),
                out_dtype=jnp.bfloat16)
    y = y[:, :4 * Cout].reshape(N, H, W, 2, 2, Cout)
    y = jnp.transpose(y, (0, 1, 3, 2, 4, 5)).reshape(N, 2 * H, 2 * W, Cout)
    return y


def kernel(e1__w1, e1__b1, e1__g1, e1__bt1, e1__w2, e1__b2, e1__g2, e1__bt2, e2__w1, e2__b1, e2__g1, e2__bt1, e2__w2, e2__b2, e2__g2, e2__bt2, e3__w1, e3__b1, e3__g1, e3__bt1, e3__w2, e3__b2, e3__g2, e3__bt2, e4__w1, e4__b1, e4__g1, e4__bt1, e4__w2, e4__b2, e4__g2, e4__bt2, b__w1, b__b1, b__g1, b__bt1, b__w2, b__b2, b__g2, b__bt2, d1__w1, d1__b1, d1__g1, d1__bt1, d1__w2, d1__b2, d1__g2, d1__bt2, d1__up_w, d1__up_b, d2__w1, d2__b1, d2__g1, d2__bt1, d2__w2, d2__b2, d2__g2, d2__bt2, d2__up_w, d2__up_b, d3__w1, d3__b1, d3__g1, d3__bt1, d3__w2, d3__b2, d3__g2, d3__bt2, d3__up_w, d3__up_b, d4__w1, d4__b1, d4__g1, d4__bt1, d4__w2, d4__b2, d4__g2, d4__bt2, d4__up_w, d4__up_b, cls_w, cls_b, x):
    p = {
        "e1": dict(w1=e1__w1, g1=e1__g1, bt1=e1__bt1,
                   w2=e1__w2, g2=e1__g2, bt2=e1__bt2),
        "e2": dict(w1=e2__w1, g1=e2__g1, bt1=e2__bt1,
                   w2=e2__w2, g2=e2__g2, bt2=e2__bt2),
        "e3": dict(w1=e3__w1, g1=e3__g1, bt1=e3__bt1,
                   w2=e3__w2, g2=e3__g2, bt2=e3__bt2),
        "e4": dict(w1=e4__w1, g1=e4__g1, bt1=e4__bt1,
                   w2=e4__w2, g2=e4__g2, bt2=e4__bt2),
        "b": dict(w1=b__w1, g1=b__g1, bt1=b__bt1,
                  w2=b__w2, g2=b__g2, bt2=b__bt2),
        "d1": dict(w1=d1__w1, g1=d1__g1, bt1=d1__bt1,
                   w2=d1__w2, g2=d1__g2, bt2=d1__bt2,
                   up_w=d1__up_w, up_b=d1__up_b),
        "d2": dict(w1=d2__w1, g1=d2__g1, bt1=d2__bt1,
                   w2=d2__w2, g2=d2__g2, bt2=d2__bt2,
                   up_w=d2__up_w, up_b=d2__up_b),
        "d3": dict(w1=d3__w1, g1=d3__g1, bt1=d3__bt1,
                   w2=d3__w2, g2=d3__g2, bt2=d3__bt2,
                   up_w=d3__up_w, up_b=d3__up_b),
        "d4": dict(w1=d4__w1, g1=d4__g1, bt1=d4__bt1,
                   w2=d4__w2, g2=d4__g2, bt2=d4__bt2,
                   up_w=d4__up_w, up_b=d4__up_b),
    }
    xin = jnp.transpose(x, (0, 2, 3, 1))

    e1, p1 = _double_conv(p["e1"], xin, first=True, pool=True)
    e2, p2 = _double_conv(p["e2"], p1, pool=True)
    e3, p3 = _double_conv(p["e3"], p2, pool=True)
    e4, p4 = _double_conv(p["e4"], p3, pool=True)
    bt = _double_conv(p["b"], p4)

    d = bt
    for name, skip in (("d1", e4), ("d2", e3), ("d3", e2), ("d4", e1)):
        u = _up_conv(d, p[name]["up_w"], p[name]["up_b"])
        d = _double_conv(p[name], [u, skip])

    N, H, W, C = d.shape
    logits = _matmul(d.reshape(N * H * W, C), cls_w, bias=cls_b)
    out = logits[:, :1].reshape(N, H, W, 1).astype(jnp.float32)
    return jnp.transpose(out, (0, 3, 1, 2))


# speedup vs baseline: 5.1066x; 1.2790x over previous
import functools

import jax
import jax.numpy as jnp
from jax.experimental import pallas as pl
from jax.experimental.pallas import tpu as pltpu

_LANE = 128
_BN_EPS = 1e-5
_VMEM_LIMIT = 56 * 1024 * 1024


def _rup(x, m):
    return (x + m - 1) // m * m


def _cparams(dim_sem):
    return pltpu.CompilerParams(dimension_semantics=dim_sem,
                                vmem_limit_bytes=_VMEM_LIMIT)


def _dconv_body(*refs, nsrc, th, w, h, bands_per_img):
    src_refs = refs[:nsrc]
    w_ref = refs[nsrc]
    y_ref = refs[nsrc + 1]
    s_ref = refs[nsrc + 2]

    i = pl.program_id(0)
    n = i // bands_per_img
    hb = i % bands_per_img
    r0 = n * (h + 2) + hb * th

    pieces = []
    for kh in range(3):
        rows = [r[pl.ds(r0 + kh, th), :, :] for r in src_refs]
        for kw in range(3):
            for rs in rows:
                pieces.append(rs[:, kw:kw + w, :])
    patch = jnp.concatenate(pieces, axis=-1)
    patch = patch.reshape(th * w, patch.shape[-1])

    y = jnp.dot(patch, w_ref[...], preferred_element_type=jnp.float32)
    y_ref[...] = y.astype(y_ref.dtype)

    s = jnp.sum(y, axis=0, keepdims=True)
    ss = jnp.sum(y * y, axis=0, keepdims=True)
    ridx = jax.lax.broadcasted_iota(jnp.int32, s_ref.shape, 0)
    s_ref[...] = jnp.where(ridx == 0, jnp.broadcast_to(s, s_ref.shape),
                           jnp.where(ridx == 1,
                                     jnp.broadcast_to(ss, s_ref.shape), 0.0))


def _direct_conv_stats(xp, nhw, w_hwio):
    N, H, W = nhw
    Cin = sum(t.shape[-1] for t in xp)
    Cout = w_hwio.shape[-1]
    M = N * H * W

    patch_cap = 10 * 1024 * 1024
    th = min(H, max(1, patch_cap // (W * 9 * Cin * 2)))
    while H % th:
        th -= 1
    bands_per_img = H // th
    nb = N * bands_per_img

    wmat = w_hwio.reshape(9 * Cin, Cout).astype(jnp.bfloat16)

    body = functools.partial(_dconv_body, nsrc=len(xp), th=th, w=W, h=H,
                             bands_per_img=bands_per_img)

    in_specs = [pl.BlockSpec(t.shape, lambda i: (0, 0, 0)) for t in xp]
    in_specs.append(pl.BlockSpec(wmat.shape, lambda i: (0, 0)))

    y, stats = pl.pallas_call(
        body,
        out_shape=(jax.ShapeDtypeStruct((M, Cout), jnp.float32),
                   jax.ShapeDtypeStruct((nb * 8, Cout), jnp.float32)),
        grid=(nb,),
        in_specs=in_specs,
        out_specs=(pl.BlockSpec((th * W, Cout), lambda i: (i, 0)),
                   pl.BlockSpec((8, Cout), lambda i: (i, 0))),
        compiler_params=_cparams(("parallel",)),
    )(*xp, wmat)
    return y, stats


def _mm_body(*refs, with_bias, with_stats, out_dtype):
    x_ref, w_ref = refs[0], refs[1]
    idx = 2 + (1 if with_bias else 0)
    y_ref = refs[idx]
    y = jnp.dot(x_ref[...], w_ref[...], preferred_element_type=jnp.float32)
    if with_bias:
        y = y + refs[2][...]
    y_ref[...] = y.astype(out_dtype)
    if with_stats:
        s_ref = refs[idx + 1]
        s = jnp.sum(y, axis=0, keepdims=True)
        ss = jnp.sum(y * y, axis=0, keepdims=True)
        ridx = jax.lax.broadcasted_iota(jnp.int32, s_ref.shape, 0)
        s_ref[...] = jnp.where(ridx == 0, jnp.broadcast_to(s, s_ref.shape),
                               jnp.where(ridx == 1,
                                         jnp.broadcast_to(ss, s_ref.shape),
                                         0.0))


def _matmul(x, wmat, bias=None, with_stats=False, out_dtype=jnp.float32):
    M, K = x.shape
    C = wmat.shape[1]
    Cp = C if C >= 64 else _rup(max(C, _LANE), _LANE)
    if Cp != C:
        wmat = jnp.pad(wmat, ((0, 0), (0, Cp - C)))
        if bias is not None:
            bias = jnp.pad(bias, ((0, Cp - C),))
    wmat = wmat.astype(jnp.bfloat16)

    if M >= 2048:
        tm = min(M // 4, max(512, (8 * 1024 * 1024) // (4 * Cp)))
    elif M >= 512:
        tm = M // 2
    else:
        tm = M
    while M % tm:
        tm -= 8
    mt = M // tm

    ops = [x.astype(jnp.bfloat16), wmat]
    in_specs = [pl.BlockSpec((tm, K), lambda i: (i, 0)),
                pl.BlockSpec((K, Cp), lambda i: (0, 0))]
    if bias is not None:
        ops.append(bias.reshape(1, Cp).astype(jnp.float32))
        in_specs.append(pl.BlockSpec((1, Cp), lambda i: (0, 0)))

    out_shape = [jax.ShapeDtypeStruct((M, Cp), out_dtype)]
    out_specs = [pl.BlockSpec((tm, Cp), lambda i: (i, 0))]
    if with_stats:
        out_shape.append(jax.ShapeDtypeStruct((mt * 8, Cp), jnp.float32))
        out_specs.append(pl.BlockSpec((8, Cp), lambda i: (i, 0)))

    body = functools.partial(_mm_body, with_bias=bias is not None,
                             with_stats=with_stats, out_dtype=out_dtype)
    res = pl.pallas_call(
        body,
        out_shape=tuple(out_shape),
        grid=(mt,),
        in_specs=in_specs,
        out_specs=tuple(out_specs),
        compiler_params=_cparams(("parallel",)),
    )(*ops)
    return res if with_stats else res[0]


def _bn_body(y_ref, s_ref, g_ref, b_ref, o_ref, *pool_ref,
             h, w, m, pad_out, pool):
    st = s_ref[...]
    ridx = jax.lax.broadcasted_iota(jnp.int32, st.shape, 0) % 8
    total = jnp.sum(jnp.where(ridx == 0, st, 0.0), axis=0, keepdims=True)
    tsq = jnp.sum(jnp.where(ridx == 1, st, 0.0), axis=0, keepdims=True)
    mean = total / m
    var = jnp.maximum(tsq / m - mean * mean, 0.0)
    inv = jax.lax.rsqrt(var + _BN_EPS)
    scale = g_ref[...] * inv
    shift = b_ref[...] - mean * scale

    o = jnp.maximum(y_ref[...] * scale + shift, 0.0).astype(jnp.bfloat16)
    c = o.shape[-1]
    if pad_out:
        o_ref[...] = jnp.pad(o.reshape(h, w, c), ((1, 1), (1, 1), (0, 0)))
    else:
        o_ref[...] = o
    if pool:
        p = jnp.max(o.reshape(h // 2, 2, w // 2, 2, c), axis=(1, 3))
        pool_ref[0][...] = jnp.pad(p, ((1, 1), (1, 1), (0, 0)))


def _bn_relu(y, stats, gamma, beta, n, h, w, pad_out=True, pool=False):
    M, C = y.shape
    body = functools.partial(_bn_body, h=h, w=w, m=M, pad_out=pad_out,
                             pool=pool)
    outs = []
    out_specs = []
    if pad_out:
        outs.append(jax.ShapeDtypeStruct((n * (h + 2), w + 2, C),
                                         jnp.bfloat16))
        out_specs.append(pl.BlockSpec((h + 2, w + 2, C),
                                      lambda i: (i, 0, 0)))
    else:
        outs.append(jax.ShapeDtypeStruct((M, C), jnp.bfloat16))
        out_specs.append(pl.BlockSpec((h * w, C), lambda i: (i, 0)))
    if pool:
        outs.append(jax.ShapeDtypeStruct((n * (h // 2 + 2), w // 2 + 2, C),
                                         jnp.bfloat16))
        out_specs.append(pl.BlockSpec((h // 2 + 2, w // 2 + 2, C),
                                      lambda i: (i, 0, 0)))
    res = pl.pallas_call(
        body,
        out_shape=tuple(outs),
        grid=(n,),
        in_specs=[pl.BlockSpec((h * w, C), lambda i: (i, 0)),
                  pl.BlockSpec(stats.shape, lambda i: (0, 0)),
                  pl.BlockSpec((1, C), lambda i: (0, 0)),
                  pl.BlockSpec((1, C), lambda i: (0, 0))],
        out_specs=tuple(out_specs),
        compiler_params=_cparams(("parallel",)),
    )(y, stats, gamma.reshape(1, C).astype(jnp.float32),
      beta.reshape(1, C).astype(jnp.float32))
    return res


def _conv_bn_relu(xps, nhw, w_hwio, gamma, beta, pad_out=True, pool=False):
    n, h, w = nhw
    y, stats = _direct_conv_stats(xps, nhw, w_hwio)
    return _bn_relu(y, stats, gamma, beta, n, h, w, pad_out=pad_out,
                    pool=pool)


def _first_conv_bn_relu(x, w_hwio, gamma, beta):
    N, H, W, Cin = x.shape
    Cout = w_hwio.shape[-1]
    M = N * H * W
    xp = jnp.pad(x.astype(jnp.bfloat16), ((0, 0), (1, 1), (1, 1), (0, 0)))
    cols = [xp[:, kh:kh + H, kw:kw + W, :]
            for kh in range(3) for kw in range(3)]
    patches = jnp.concatenate(cols, axis=-1).reshape(M, 9 * Cin)
    y, stats = _matmul(patches, w_hwio.reshape(9 * Cin, Cout),
                       with_stats=True)
    return _bn_relu(y, stats, gamma, beta, N, H, W, pad_out=True)[0]


def _up_body(x_ref, w_ref, b_ref, o_ref, *, h, w, cout):
    y = jnp.dot(x_ref[...], w_ref[...],
                preferred_element_type=jnp.float32) + b_ref[...]
    y = y.astype(jnp.bfloat16)
    q = [y[:, d * cout:(d + 1) * cout].reshape(h, w, 1, cout)
         for d in range(4)]
    top = jnp.concatenate(q[0:2], axis=2).reshape(h, 1, 2 * w, cout)
    bot = jnp.concatenate(q[2:4], axis=2).reshape(h, 1, 2 * w, cout)
    full = jnp.concatenate([top, bot], axis=1).reshape(2 * h, 2 * w, cout)
    o_ref[...] = jnp.pad(full, ((1, 1), (1, 1), (0, 0)))


def _up_conv(x, w, b, nhw):
    n, h, wd = nhw
    Cin = x.shape[-1]
    Cout = w.shape[-1]
    wmat = w.reshape(Cin, 4 * Cout).astype(jnp.bfloat16)
    bias4 = jnp.tile(b, 4).reshape(1, 4 * Cout).astype(jnp.float32)
    return pl.pallas_call(
        functools.partial(_up_body, h=h, w=wd, cout=Cout),
        out_shape=jax.ShapeDtypeStruct((n * (2 * h + 2), 2 * wd + 2, Cout),
                                       jnp.bfloat16),
        grid=(n,),
        in_specs=[pl.BlockSpec((h * wd, Cin), lambda i: (i, 0)),
                  pl.BlockSpec((Cin, 4 * Cout), lambda i: (0, 0)),
                  pl.BlockSpec((1, 4 * Cout), lambda i: (0, 0))],
        out_specs=pl.BlockSpec((2 * h + 2, 2 * wd + 2, Cout),
                               lambda i: (i, 0, 0)),
        compiler_params=_cparams(("parallel",)),
    )(x.astype(jnp.bfloat16), wmat, bias4)


def kernel(e1__w1, e1__b1, e1__g1, e1__bt1, e1__w2, e1__b2, e1__g2, e1__bt2, e2__w1, e2__b1, e2__g1, e2__bt1, e2__w2, e2__b2, e2__g2, e2__bt2, e3__w1, e3__b1, e3__g1, e3__bt1, e3__w2, e3__b2, e3__g2, e3__bt2, e4__w1, e4__b1, e4__g1, e4__bt1, e4__w2, e4__b2, e4__g2, e4__bt2, b__w1, b__b1, b__g1, b__bt1, b__w2, b__b2, b__g2, b__bt2, d1__w1, d1__b1, d1__g1, d1__bt1, d1__w2, d1__b2, d1__g2, d1__bt2, d1__up_w, d1__up_b, d2__w1, d2__b1, d2__g1, d2__bt1, d2__w2, d2__b2, d2__g2, d2__bt2, d2__up_w, d2__up_b, d3__w1, d3__b1, d3__g1, d3__bt1, d3__w2, d3__b2, d3__g2, d3__bt2, d3__up_w, d3__up_b, d4__w1, d4__b1, d4__g1, d4__bt1, d4__w2, d4__b2, d4__g2, d4__bt2, d4__up_w, d4__up_b, cls_w, cls_b, x):
    p = {
        "e1": dict(w1=e1__w1, g1=e1__g1, bt1=e1__bt1,
                   w2=e1__w2, g2=e1__g2, bt2=e1__bt2),
        "e2": dict(w1=e2__w1, g1=e2__g1, bt1=e2__bt1,
                   w2=e2__w2, g2=e2__g2, bt2=e2__bt2),
        "e3": dict(w1=e3__w1, g1=e3__g1, bt1=e3__bt1,
                   w2=e3__w2, g2=e3__g2, bt2=e3__bt2),
        "e4": dict(w1=e4__w1, g1=e4__g1, bt1=e4__bt1,
                   w2=e4__w2, g2=e4__g2, bt2=e4__bt2),
        "b": dict(w1=b__w1, g1=b__g1, bt1=b__bt1,
                  w2=b__w2, g2=b__g2, bt2=b__bt2),
        "d1": dict(w1=d1__w1, g1=d1__g1, bt1=d1__bt1,
                   w2=d1__w2, g2=d1__g2, bt2=d1__bt2,
                   up_w=d1__up_w, up_b=d1__up_b),
        "d2": dict(w1=d2__w1, g1=d2__g1, bt1=d2__bt1,
                   w2=d2__w2, g2=d2__g2, bt2=d2__bt2,
                   up_w=d2__up_w, up_b=d2__up_b),
        "d3": dict(w1=d3__w1, g1=d3__g1, bt1=d3__bt1,
                   w2=d3__w2, g2=d3__g2, bt2=d3__bt2,
                   up_w=d3__up_w, up_b=d3__up_b),
        "d4": dict(w1=d4__w1, g1=d4__g1, bt1=d4__bt1,
                   w2=d4__w2, g2=d4__g2, bt2=d4__bt2,
                   up_w=d4__up_w, up_b=d4__up_b),
    }
    xin = jnp.transpose(x, (0, 2, 3, 1))
    N = xin.shape[0]

    cur = _first_conv_bn_relu(xin, p["e1"]["w1"], p["e1"]["g1"],
                              p["e1"]["bt1"])
    skips = {}
    hw = xin.shape[1]
    for name in ("e1", "e2", "e3", "e4"):
        q = p[name]
        if name != "e1":
            cur = _conv_bn_relu([cur], (N, hw, hw), q["w1"], q["g1"],
                                q["bt1"])[0]
        full, pooled = _conv_bn_relu([cur], (N, hw, hw), q["w2"], q["g2"],
                                     q["bt2"], pool=True)
        skips[name] = full
        cur = pooled
        hw //= 2

    cur = _conv_bn_relu([cur], (N, hw, hw), p["b"]["w1"], p["b"]["g1"],
                        p["b"]["bt1"])[0]
    flat = _conv_bn_relu([cur], (N, hw, hw), p["b"]["w2"], p["b"]["g2"],
                         p["b"]["bt2"], pad_out=False)[0]

    for name, skip in (("d1", "e4"), ("d2", "e3"), ("d3", "e2"),
                       ("d4", "e1")):
        q = p[name]
        u = _up_conv(flat, q["up_w"], q["up_b"], (N, hw, hw))
        hw *= 2
        cur = _conv_bn_relu([u, skips[skip]], (N, hw, hw), q["w1"],
                            q["g1"], q["bt1"])[0]
        flat = _conv_bn_relu([cur], (N, hw, hw), q["w2"], q["g2"],
                             q["bt2"], pad_out=False)[0]

    logits = _matmul(flat, cls_w, bias=cls_b)
    out = logits[:, :1].reshape(N, hw, hw, 1).astype(jnp.float32)
    return jnp.transpose(out, (0, 3, 1, 2))


# speedup vs baseline: 5.1088x; 1.0004x over previous
import functools

import jax
import jax.numpy as jnp
from jax.experimental import pallas as pl
from jax.experimental.pallas import tpu as pltpu

_LANE = 128
_BN_EPS = 1e-5
_VMEM_LIMIT = 56 * 1024 * 1024


def _rup(x, m):
    return (x + m - 1) // m * m


def _cparams(dim_sem):
    return pltpu.CompilerParams(dimension_semantics=dim_sem,
                                vmem_limit_bytes=_VMEM_LIMIT)


def _dconv_body(*refs, nsrc, th, w, h, bands_per_img):
    src_refs = refs[:nsrc]
    w_ref = refs[nsrc]
    y_ref = refs[nsrc + 1]
    s_ref = refs[nsrc + 2]

    i = pl.program_id(0)
    n = i // bands_per_img
    hb = i % bands_per_img
    r0 = n * (h + 2) + hb * th

    pieces = []
    for kh in range(3):
        rows = [r[pl.ds(r0 + kh, th), :, :] for r in src_refs]
        for kw in range(3):
            for rs in rows:
                pieces.append(rs[:, kw:kw + w, :])
    patch = jnp.concatenate(pieces, axis=-1)
    patch = patch.reshape(th * w, patch.shape[-1])

    y = jnp.dot(patch, w_ref[...], preferred_element_type=jnp.float32)
    y_ref[...] = y.astype(y_ref.dtype)

    s = jnp.sum(y, axis=0, keepdims=True)
    ss = jnp.sum(y * y, axis=0, keepdims=True)
    ridx = jax.lax.broadcasted_iota(jnp.int32, s_ref.shape, 0)
    s_ref[...] = jnp.where(ridx == 0, jnp.broadcast_to(s, s_ref.shape),
                           jnp.where(ridx == 1,
                                     jnp.broadcast_to(ss, s_ref.shape), 0.0))


def _direct_conv_stats(xp, nhw, w_hwio):
    N, H, W = nhw
    Cin = sum(t.shape[-1] for t in xp)
    Cout = w_hwio.shape[-1]
    M = N * H * W

    patch_cap = 16 * 1024 * 1024
    th = min(H, max(1, patch_cap // (W * 9 * Cin * 2)))
    while H % th:
        th -= 1
    bands_per_img = H // th
    nb = N * bands_per_img

    wmat = w_hwio.reshape(9 * Cin, Cout).astype(jnp.bfloat16)

    body = functools.partial(_dconv_body, nsrc=len(xp), th=th, w=W, h=H,
                             bands_per_img=bands_per_img)

    in_specs = [pl.BlockSpec(t.shape, lambda i: (0, 0, 0)) for t in xp]
    in_specs.append(pl.BlockSpec(wmat.shape, lambda i: (0, 0)))

    y, stats = pl.pallas_call(
        body,
        out_shape=(jax.ShapeDtypeStruct((M, Cout), jnp.float32),
                   jax.ShapeDtypeStruct((nb * 8, Cout), jnp.float32)),
        grid=(nb,),
        in_specs=in_specs,
        out_specs=(pl.BlockSpec((th * W, Cout), lambda i: (i, 0)),
                   pl.BlockSpec((8, Cout), lambda i: (i, 0))),
        compiler_params=_cparams(("parallel",)),
    )(*xp, wmat)
    return y, stats


def _mm_body(*refs, with_bias, with_stats, out_dtype):
    x_ref, w_ref = refs[0], refs[1]
    idx = 2 + (1 if with_bias else 0)
    y_ref = refs[idx]
    y = jnp.dot(x_ref[...], w_ref[...], preferred_element_type=jnp.float32)
    if with_bias:
        y = y + refs[2][...]
    y_ref[...] = y.astype(out_dtype)
    if with_stats:
        s_ref = refs[idx + 1]
        s = jnp.sum(y, axis=0, keepdims=True)
        ss = jnp.sum(y * y, axis=0, keepdims=True)
        ridx = jax.lax.broadcasted_iota(jnp.int32, s_ref.shape, 0)
        s_ref[...] = jnp.where(ridx == 0, jnp.broadcast_to(s, s_ref.shape),
                               jnp.where(ridx == 1,
                                         jnp.broadcast_to(ss, s_ref.shape),
                                         0.0))


def _matmul(x, wmat, bias=None, with_stats=False, out_dtype=jnp.float32):
    M, K = x.shape
    C = wmat.shape[1]
    Cp = C if C >= 64 else _rup(max(C, _LANE), _LANE)
    if Cp != C:
        wmat = jnp.pad(wmat, ((0, 0), (0, Cp - C)))
        if bias is not None:
            bias = jnp.pad(bias, ((0, Cp - C),))
    wmat = wmat.astype(jnp.bfloat16)

    if M >= 2048:
        tm = min(M // 4, max(512, (8 * 1024 * 1024) // (4 * Cp)))
    elif M >= 512:
        tm = M // 2
    else:
        tm = M
    while M % tm:
        tm -= 8
    mt = M // tm

    ops = [x.astype(jnp.bfloat16), wmat]
    in_specs = [pl.BlockSpec((tm, K), lambda i: (i, 0)),
                pl.BlockSpec((K, Cp), lambda i: (0, 0))]
    if bias is not None:
        ops.append(bias.reshape(1, Cp).astype(jnp.float32))
        in_specs.append(pl.BlockSpec((1, Cp), lambda i: (0, 0)))

    out_shape = [jax.ShapeDtypeStruct((M, Cp), out_dtype)]
    out_specs = [pl.BlockSpec((tm, Cp), lambda i: (i, 0))]
    if with_stats:
        out_shape.append(jax.ShapeDtypeStruct((mt * 8, Cp), jnp.float32))
        out_specs.append(pl.BlockSpec((8, Cp), lambda i: (i, 0)))

    body = functools.partial(_mm_body, with_bias=bias is not None,
                             with_stats=with_stats, out_dtype=out_dtype)
    res = pl.pallas_call(
        body,
        out_shape=tuple(out_shape),
        grid=(mt,),
        in_specs=in_specs,
        out_specs=tuple(out_specs),
        compiler_params=_cparams(("parallel",)),
    )(*ops)
    return res if with_stats else res[0]


def _bn_body(y_ref, s_ref, g_ref, b_ref, o_ref, *pool_ref,
             h, w, m, pad_out, pool):
    st = s_ref[...]
    ridx = jax.lax.broadcasted_iota(jnp.int32, st.shape, 0) % 8
    total = jnp.sum(jnp.where(ridx == 0, st, 0.0), axis=0, keepdims=True)
    tsq = jnp.sum(jnp.where(ridx == 1, st, 0.0), axis=0, keepdims=True)
    mean = total / m
    var = jnp.maximum(tsq / m - mean * mean, 0.0)
    inv = jax.lax.rsqrt(var + _BN_EPS)
    scale = g_ref[...] * inv
    shift = b_ref[...] - mean * scale

    o = jnp.maximum(y_ref[...] * scale + shift, 0.0).astype(jnp.bfloat16)
    c = o.shape[-1]
    if pad_out:
        o_ref[...] = jnp.pad(o.reshape(h, w, c), ((1, 1), (1, 1), (0, 0)))
    else:
        o_ref[...] = o
    if pool:
        p = jnp.max(o.reshape(h // 2, 2, w // 2, 2, c), axis=(1, 3))
        pool_ref[0][...] = jnp.pad(p, ((1, 1), (1, 1), (0, 0)))


def _bn_relu(y, stats, gamma, beta, n, h, w, pad_out=True, pool=False):
    M, C = y.shape
    body = functools.partial(_bn_body, h=h, w=w, m=M, pad_out=pad_out,
                             pool=pool)
    outs = []
    out_specs = []
    if pad_out:
        outs.append(jax.ShapeDtypeStruct((n * (h + 2), w + 2, C),
                                         jnp.bfloat16))
        out_specs.append(pl.BlockSpec((h + 2, w + 2, C),
                                      lambda i: (i, 0, 0)))
    else:
        outs.append(jax.ShapeDtypeStruct((M, C), jnp.bfloat16))
        out_specs.append(pl.BlockSpec((h * w, C), lambda i: (i, 0)))
    if pool:
        outs.append(jax.ShapeDtypeStruct((n * (h // 2 + 2), w // 2 + 2, C),
                                         jnp.bfloat16))
        out_specs.append(pl.BlockSpec((h // 2 + 2, w // 2 + 2, C),
                                      lambda i: (i, 0, 0)))
    res = pl.pallas_call(
        body,
        out_shape=tuple(outs),
        grid=(n,),
        in_specs=[pl.BlockSpec((h * w, C), lambda i: (i, 0)),
                  pl.BlockSpec(stats.shape, lambda i: (0, 0)),
                  pl.BlockSpec((1, C), lambda i: (0, 0)),
                  pl.BlockSpec((1, C), lambda i: (0, 0))],
        out_specs=tuple(out_specs),
        compiler_params=_cparams(("parallel",)),
    )(y, stats, gamma.reshape(1, C).astype(jnp.float32),
      beta.reshape(1, C).astype(jnp.float32))
    return res


def _conv_bn_relu(xps, nhw, w_hwio, gamma, beta, pad_out=True, pool=False):
    n, h, w = nhw
    y, stats = _direct_conv_stats(xps, nhw, w_hwio)
    return _bn_relu(y, stats, gamma, beta, n, h, w, pad_out=pad_out,
                    pool=pool)


def _first_conv_bn_relu(x, w_hwio, gamma, beta):
    N, H, W, Cin = x.shape
    Cout = w_hwio.shape[-1]
    M = N * H * W
    xp = jnp.pad(x.astype(jnp.bfloat16), ((0, 0), (1, 1), (1, 1), (0, 0)))
    cols = [xp[:, kh:kh + H, kw:kw + W, :]
            for kh in range(3) for kw in range(3)]
    patches = jnp.concatenate(cols, axis=-1).reshape(M, 9 * Cin)
    y, stats = _matmul(patches, w_hwio.reshape(9 * Cin, Cout),
                       with_stats=True)
    return _bn_relu(y, stats, gamma, beta, N, H, W, pad_out=True)[0]


def _up_body(x_ref, w_ref, b_ref, o_ref, *, h, w, cout):
    y = jnp.dot(x_ref[...], w_ref[...],
                preferred_element_type=jnp.float32) + b_ref[...]
    y = y.astype(jnp.bfloat16)
    q = [y[:, d * cout:(d + 1) * cout].reshape(h, w, 1, cout)
         for d in range(4)]
    top = jnp.concatenate(q[0:2], axis=2).reshape(h, 1, 2 * w, cout)
    bot = jnp.concatenate(q[2:4], axis=2).reshape(h, 1, 2 * w, cout)
    full = jnp.concatenate([top, bot], axis=1).reshape(2 * h, 2 * w, cout)
    o_ref[...] = jnp.pad(full, ((1, 1), (1, 1), (0, 0)))


def _up_conv(x, w, b, nhw):
    n, h, wd = nhw
    Cin = x.shape[-1]
    Cout = w.shape[-1]
    wmat = w.reshape(Cin, 4 * Cout).astype(jnp.bfloat16)
    bias4 = jnp.tile(b, 4).reshape(1, 4 * Cout).astype(jnp.float32)
    return pl.pallas_call(
        functools.partial(_up_body, h=h, w=wd, cout=Cout),
        out_shape=jax.ShapeDtypeStruct((n * (2 * h + 2), 2 * wd + 2, Cout),
                                       jnp.bfloat16),
        grid=(n,),
        in_specs=[pl.BlockSpec((h * wd, Cin), lambda i: (i, 0)),
                  pl.BlockSpec((Cin, 4 * Cout), lambda i: (0, 0)),
                  pl.BlockSpec((1, 4 * Cout), lambda i: (0, 0))],
        out_specs=pl.BlockSpec((2 * h + 2, 2 * wd + 2, Cout),
                               lambda i: (i, 0, 0)),
        compiler_params=_cparams(("parallel",)),
    )(x.astype(jnp.bfloat16), wmat, bias4)


def kernel(e1__w1, e1__b1, e1__g1, e1__bt1, e1__w2, e1__b2, e1__g2, e1__bt2, e2__w1, e2__b1, e2__g1, e2__bt1, e2__w2, e2__b2, e2__g2, e2__bt2, e3__w1, e3__b1, e3__g1, e3__bt1, e3__w2, e3__b2, e3__g2, e3__bt2, e4__w1, e4__b1, e4__g1, e4__bt1, e4__w2, e4__b2, e4__g2, e4__bt2, b__w1, b__b1, b__g1, b__bt1, b__w2, b__b2, b__g2, b__bt2, d1__w1, d1__b1, d1__g1, d1__bt1, d1__w2, d1__b2, d1__g2, d1__bt2, d1__up_w, d1__up_b, d2__w1, d2__b1, d2__g1, d2__bt1, d2__w2, d2__b2, d2__g2, d2__bt2, d2__up_w, d2__up_b, d3__w1, d3__b1, d3__g1, d3__bt1, d3__w2, d3__b2, d3__g2, d3__bt2, d3__up_w, d3__up_b, d4__w1, d4__b1, d4__g1, d4__bt1, d4__w2, d4__b2, d4__g2, d4__bt2, d4__up_w, d4__up_b, cls_w, cls_b, x):
    p = {
        "e1": dict(w1=e1__w1, g1=e1__g1, bt1=e1__bt1,
                   w2=e1__w2, g2=e1__g2, bt2=e1__bt2),
        "e2": dict(w1=e2__w1, g1=e2__g1, bt1=e2__bt1,
                   w2=e2__w2, g2=e2__g2, bt2=e2__bt2),
        "e3": dict(w1=e3__w1, g1=e3__g1, bt1=e3__bt1,
                   w2=e3__w2, g2=e3__g2, bt2=e3__bt2),
        "e4": dict(w1=e4__w1, g1=e4__g1, bt1=e4__bt1,
                   w2=e4__w2, g2=e4__g2, bt2=e4__bt2),
        "b": dict(w1=b__w1, g1=b__g1, bt1=b__bt1,
                  w2=b__w2, g2=b__g2, bt2=b__bt2),
        "d1": dict(w1=d1__w1, g1=d1__g1, bt1=d1__bt1,
                   w2=d1__w2, g2=d1__g2, bt2=d1__bt2,
                   up_w=d1__up_w, up_b=d1__up_b),
        "d2": dict(w1=d2__w1, g1=d2__g1, bt1=d2__bt1,
                   w2=d2__w2, g2=d2__g2, bt2=d2__bt2,
                   up_w=d2__up_w, up_b=d2__up_b),
        "d3": dict(w1=d3__w1, g1=d3__g1, bt1=d3__bt1,
                   w2=d3__w2, g2=d3__g2, bt2=d3__bt2,
                   up_w=d3__up_w, up_b=d3__up_b),
        "d4": dict(w1=d4__w1, g1=d4__g1, bt1=d4__bt1,
                   w2=d4__w2, g2=d4__g2, bt2=d4__bt2,
                   up_w=d4__up_w, up_b=d4__up_b),
    }
    xin = jnp.transpose(x, (0, 2, 3, 1))
    N = xin.shape[0]

    cur = _first_conv_bn_relu(xin, p["e1"]["w1"], p["e1"]["g1"],
                              p["e1"]["bt1"])
    skips = {}
    hw = xin.shape[1]
    for name in ("e1", "e2", "e3", "e4"):
        q = p[name]
        if name != "e1":
            cur = _conv_bn_relu([cur], (N, hw, hw), q["w1"], q["g1"],
                                q["bt1"])[0]
        full, pooled = _conv_bn_relu([cur], (N, hw, hw), q["w2"], q["g2"],
                                     q["bt2"], pool=True)
        skips[name] = full
        cur = pooled
        hw //= 2

    cur = _conv_bn_relu([cur], (N, hw, hw), p["b"]["w1"], p["b"]["g1"],
                        p["b"]["bt1"])[0]
    flat = _conv_bn_relu([cur], (N, hw, hw), p["b"]["w2"], p["b"]["g2"],
                         p["b"]["bt2"], pad_out=False)[0]

    for name, skip in (("d1", "e4"), ("d2", "e3"), ("d3", "e2"),
                       ("d4", "e1")):
        q = p[name]
        u = _up_conv(flat, q["up_w"], q["up_b"], (N, hw, hw))
        hw *= 2
        cur = _conv_bn_relu([u, skips[skip]], (N, hw, hw), q["w1"],
                            q["g1"], q["bt1"])[0]
        flat = _conv_bn_relu([cur], (N, hw, hw), q["w2"], q["g2"],
                             q["bt2"], pad_out=False)[0]

    logits = _matmul(flat, cls_w, bias=cls_b)
    out = logits[:, :1].reshape(N, hw, hw, 1).astype(jnp.float32)
    return jnp.transpose(out, (0, 3, 1, 2))
